# Initial kernel scaffold; baseline (speedup 1.0000x reference)
import jax
import jax.numpy as jnp
from jax.experimental import pallas as pl


def kernel(xb, noise):
    raise NotImplementedError("write your pallas kernel here")


# speedup vs baseline: 1.9127x; 1.9127x over previous
import functools

import jax
import jax.numpy as jnp
from jax.experimental import pallas as pl
from jax.experimental.pallas import tpu as pltpu

_MASK_RATIO = 0.4


def _mask_kernel(len_keep, l_total, noise_ref, mask_ref):
    bits = jax.lax.bitcast_convert_type(noise_ref[...], jnp.int32)
    ncols = bits.shape[1]
    k = jnp.int32(len_keep)

    t = jnp.zeros((1, ncols), jnp.int32)
    for b in range(29, -1, -1):
        cand = t + jnp.int32(1 << b)
        cnt = jnp.sum((bits < cand).astype(jnp.int32), axis=0, keepdims=True)
        t = jnp.where(cnt < k, cand, t)

    below = (bits < t)
    n_below = jnp.sum(below.astype(jnp.int32), axis=0, keepdims=True)
    n_tie_keep = k - n_below

    eq = (bits == t).astype(jnp.float32)
    row = jax.lax.broadcasted_iota(jnp.int32, (l_total, l_total), 0)
    col = jax.lax.broadcasted_iota(jnp.int32, (l_total, l_total), 1)
    tril = (col < row).astype(jnp.float32)
    eq_before = jax.lax.dot_general(
        tril, eq, (((1,), (0,)), ((), ())),
        preferred_element_type=jnp.float32)

    keep = below | (eq > 0.5) & (eq_before < n_tie_keep.astype(jnp.float32))
    mask_ref[...] = keep.astype(jnp.float32)


def _apply_kernel(xb_ref, mask_ref, out_ref):
    out_ref[...] = xb_ref[...] * mask_ref[...]


@jax.jit
def kernel(xb, noise):
    bs, L, nvars, D = xb.shape
    len_keep = int(L * (1 - _MASK_RATIO))
    C = bs * nvars

    noise_t = jnp.transpose(noise, (1, 0, 2)).reshape(L, C)

    mask = pl.pallas_call(
        functools.partial(_mask_kernel, len_keep, L),
        out_shape=jax.ShapeDtypeStruct((L, C), jnp.float32),
    )(noise_t)

    mask_b = jnp.transpose(mask.reshape(L, bs, nvars), (1, 0, 2))[..., None]

    Lc = 128
    grid = (bs, L // Lc)
    out = pl.pallas_call(
        _apply_kernel,
        grid=grid,
        in_specs=[
            pl.BlockSpec((1, Lc, nvars, D), lambda b, l: (b, l, 0, 0)),
            pl.BlockSpec((1, Lc, nvars, 1), lambda b, l: (b, l, 0, 0)),
        ],
        out_specs=pl.BlockSpec((1, Lc, nvars, D), lambda b, l: (b, l, 0, 0)),
        out_shape=jax.ShapeDtypeStruct((bs, L, nvars, D), jnp.float32),
    )(xb, mask_b)
    return out


# speedup vs baseline: 2.7729x; 1.4497x over previous
import functools

import jax
import jax.numpy as jnp
from jax.experimental import pallas as pl

_MASK_RATIO = 0.4


def _mask_kernel(len_keep, l_total, bs, nvars, noise_ref, mask_ref):
    bits = jax.lax.bitcast_convert_type(noise_ref[...], jnp.int32)
    ncols = bits.shape[0]
    k = jnp.int32(len_keep)

    t = jnp.zeros((ncols, 1), jnp.int32)
    for b in range(29, -1, -1):
        cand = t + jnp.int32(1 << b)
        cnt = jnp.sum((bits < cand).astype(jnp.int32), axis=1, keepdims=True)
        t = jnp.where(cnt < k, cand, t)

    below = (bits < t)
    n_below = jnp.sum(below.astype(jnp.int32), axis=1, keepdims=True)
    n_tie_keep = k - n_below

    eq = (bits == t).astype(jnp.float32)
    row = jax.lax.broadcasted_iota(jnp.int32, (l_total, l_total), 0)
    col = jax.lax.broadcasted_iota(jnp.int32, (l_total, l_total), 1)
    ut = (row < col).astype(jnp.float32)
    eq_before = jax.lax.dot_general(
        eq, ut, (((1,), (0,)), ((), ())),
        preferred_element_type=jnp.float32)

    keep = below | (eq > 0.5) & (eq_before < n_tie_keep.astype(jnp.float32))
    mask_ref[...] = keep.astype(jnp.float32).reshape(bs, nvars, l_total)


def _apply_kernel(xb_ref, mask_ref, out_ref):
    m = mask_ref[0]
    mt = jnp.transpose(m)
    out_ref[...] = xb_ref[...] * mt[None, :, :, None]


@jax.jit
def kernel(xb, noise):
    bs, L, nvars, D = xb.shape
    len_keep = int(L * (1 - _MASK_RATIO))
    C = bs * nvars

    noise_r = jnp.transpose(noise, (0, 2, 1)).reshape(C, L)

    mask = pl.pallas_call(
        functools.partial(_mask_kernel, len_keep, L, bs, nvars),
        out_shape=jax.ShapeDtypeStruct((bs, nvars, L), jnp.float32),
    )(noise_r)

    grid = (bs,)
    out = pl.pallas_call(
        _apply_kernel,
        grid=grid,
        in_specs=[
            pl.BlockSpec((1, L, nvars, D), lambda b: (b, 0, 0, 0)),
            pl.BlockSpec((1, nvars, L), lambda b: (b, 0, 0)),
        ],
        out_specs=pl.BlockSpec((1, L, nvars, D), lambda b: (b, 0, 0, 0)),
        out_shape=jax.ShapeDtypeStruct((bs, L, nvars, D), jnp.float32),
    )(xb, mask)
    return out
